# baseline (speedup 1.0000x reference)
import functools
import jax
import jax.numpy as jnp
from jax import lax
from jax.experimental import pallas as pl
from jax.experimental.pallas import tpu as pltpu
from jax.experimental.pallas import tpu_sc as plsc

_L = 50
_B = 4096
_D = 128
_R = 6
_NW = 32
_CHUNK = _B * _D // _NW


def _tc_body(idx_ref, x_ref, buf_ref, nbuf_ref):
    i = pl.program_id(0)
    r = idx_ref[0] - _R * i
    nbuf_ref[...] = buf_ref[...]
    hit = jnp.logical_and(r >= 0, r < _R)

    @pl.when(hit)
    def _write_slot():
        nbuf_ref[r] = x_ref[...]


def _tc_rewrite(idx, x, buffer):
    grid_spec = pltpu.PrefetchScalarGridSpec(
        num_scalar_prefetch=1,
        grid=(pl.cdiv(_L, _R),),
        in_specs=[
            pl.BlockSpec((_B, _D), lambda i, *_: (0, 0)),
            pl.BlockSpec((_R, _B, _D), lambda i, *_: (i, 0, 0)),
        ],
        out_specs=pl.BlockSpec((_R, _B, _D), lambda i, *_: (i, 0, 0)),
    )
    return pl.pallas_call(
        _tc_body,
        grid_spec=grid_spec,
        out_shape=jax.ShapeDtypeStruct((_L, _B, _D), buffer.dtype),
    )(idx, x, buffer)


def _sc_body(base16_hbm, cc16_hbm, bufv_hbm, outv_hbm,
             base_v, cc_v, row_v, sem):
    c = lax.axis_index("c")
    s = lax.axis_index("s")
    wid = s * 2 + c
    pltpu.sync_copy(base16_hbm, base_v)
    pltpu.sync_copy(cc16_hbm, cc_v)
    base = jnp.max(base_v[...])
    ccs = jnp.max(cc_v[...])
    pltpu.async_copy(bufv_hbm.at[pl.ds(base + wid, 1)], row_v, sem).wait()

    @pl.when(ccs < _L)
    def _zero():
        def _step(j, carry):
            row_v[0, pl.ds(j * 16, 16)] = jnp.zeros((16,), jnp.float32)
            return carry
        lax.fori_loop(0, _CHUNK // 16, _step, 0)

    pltpu.sync_copy(row_v, outv_hbm.at[pl.ds(wid, 1)])


def _sc_gather(base16, cc16, bufv):
    run = functools.partial(
        pl.kernel,
        out_type=jax.ShapeDtypeStruct((_NW, _CHUNK), jnp.float32),
        mesh=plsc.VectorSubcoreMesh(core_axis_name="c", subcore_axis_name="s"),
        scratch_types=[
            pltpu.VMEM((16,), jnp.int32),
            pltpu.VMEM((16,), jnp.int32),
            pltpu.VMEM((1, _CHUNK), jnp.float32),
            pltpu.SemaphoreType.DMA,
        ],
        compiler_params=pltpu.CompilerParams(needs_layout_passes=False),
    )(_sc_body)
    return run(base16, cc16, bufv)


def kernel(x, buffer, index, call_count):
    idx = jnp.asarray(index, jnp.int32).reshape(1)
    base16 = jnp.full((16,), jnp.asarray(index, jnp.int32) * _NW, jnp.int32)
    cc16 = jnp.full((16,), jnp.asarray(call_count, jnp.int32), jnp.int32)
    bufv = buffer.reshape(_L * _NW, _CHUNK)
    outv = _sc_gather(base16, cc16, bufv)
    new_buffer = _tc_rewrite(idx, x, buffer)
    return outv.reshape(_B, _D), new_buffer


# speedup vs baseline: 2.3524x; 2.3524x over previous
import functools
import jax
import jax.numpy as jnp
from jax import lax
from jax.experimental import pallas as pl
from jax.experimental.pallas import tpu as pltpu
from jax.experimental.pallas import tpu_sc as plsc

_L = 50
_B = 4096
_D = 128
_R = 6
_NW = 32
_CHUNK = _B * _D // _NW


def _tc_body(idx_ref, x_ref, buf_ref, nbuf_ref):
    i = pl.program_id(0)
    r = idx_ref[0] - _R * i
    nbuf_ref[...] = buf_ref[...]
    hit = jnp.logical_and(r >= 0, r < _R)

    @pl.when(hit)
    def _write_slot():
        nbuf_ref[r] = x_ref[...]


def _tc_rewrite(idx, x, buffer):
    grid_spec = pltpu.PrefetchScalarGridSpec(
        num_scalar_prefetch=1,
        grid=(pl.cdiv(_L, _R),),
        in_specs=[
            pl.BlockSpec((_B, _D), lambda i, *_: (0, 0)),
            pl.BlockSpec((_R, _B, _D), lambda i, *_: (i, 0, 0)),
        ],
        out_specs=pl.BlockSpec((_R, _B, _D), lambda i, *_: (i, 0, 0)),
    )
    return pl.pallas_call(
        _tc_body,
        grid_spec=grid_spec,
        out_shape=jax.ShapeDtypeStruct((_L, _B, _D), buffer.dtype),
    )(idx, x, buffer)


_BW = _B // _NW


def _sc_body(base16_hbm, cc16_hbm, bufv_hbm, out_hbm,
             base_v, cc_v, slab_v, sem):
    c = lax.axis_index("c")
    s = lax.axis_index("s")
    wid = s * 2 + c
    pltpu.sync_copy(base16_hbm, base_v)
    pltpu.sync_copy(cc16_hbm, cc_v)
    base = jnp.max(base_v[...])
    ccs = jnp.max(cc_v[...])
    start = pl.multiple_of(base + wid * _BW, _BW)
    pltpu.async_copy(bufv_hbm.at[pl.ds(start, _BW)], slab_v, sem).wait()

    @pl.when(ccs < _L)
    def _zero():
        def _step(j, carry):
            slab_v[j // 8, pl.ds((j % 8) * 16, 16)] = (
                jnp.zeros((16,), jnp.float32))
            return carry
        lax.fori_loop(0, _BW * _D // 16, _step, 0)

    pltpu.sync_copy(slab_v, out_hbm.at[pl.ds(wid * _BW, _BW)])


def _sc_gather(base16, cc16, bufv):
    run = functools.partial(
        pl.kernel,
        out_type=jax.ShapeDtypeStruct((_B, _D), jnp.float32),
        mesh=plsc.VectorSubcoreMesh(core_axis_name="c", subcore_axis_name="s"),
        scratch_types=[
            pltpu.VMEM((16,), jnp.int32),
            pltpu.VMEM((16,), jnp.int32),
            pltpu.VMEM((_BW, _D), jnp.float32),
            pltpu.SemaphoreType.DMA,
        ],
        compiler_params=pltpu.CompilerParams(needs_layout_passes=False),
    )(_sc_body)
    return run(base16, cc16, bufv)


def kernel(x, buffer, index, call_count):
    idx = jnp.asarray(index, jnp.int32).reshape(1)
    base16 = jnp.full((16,), jnp.asarray(index, jnp.int32) * _B, jnp.int32)
    cc16 = jnp.full((16,), jnp.asarray(call_count, jnp.int32), jnp.int32)
    bufv = buffer.reshape(_L * _B, _D)
    output = _sc_gather(base16, cc16, bufv)
    new_buffer = _tc_rewrite(idx, x, buffer)
    return output, new_buffer


# speedup vs baseline: 2.9019x; 1.2336x over previous
import jax
import jax.numpy as jnp
from jax.experimental import pallas as pl
from jax.experimental.pallas import tpu as pltpu

_L = 50
_B = 4096
_D = 128
_R = 6
_NJ = 2
_BJ = _B // _NJ


def _body(idx_ref, cc_ref, x_ref, buf_ref, out_ref, nbuf_ref):
    i = pl.program_id(1)
    r = idx_ref[0] - _R * i
    nbuf_ref[...] = buf_ref[...]
    hit = jnp.logical_and(r >= 0, r < _R)

    @pl.when(hit)
    def _write_slot():
        nbuf_ref[r] = x_ref[...]
        out_ref[...] = jnp.where(cc_ref[0] >= _L, buf_ref[r],
                                 jnp.zeros_like(x_ref))


def kernel(x, buffer, index, call_count):
    idx = jnp.asarray(index, jnp.int32).reshape(1)
    cc = jnp.asarray(call_count, jnp.int32).reshape(1)
    grid_spec = pltpu.PrefetchScalarGridSpec(
        num_scalar_prefetch=2,
        grid=(_NJ, pl.cdiv(_L, _R)),
        in_specs=[
            pl.BlockSpec((_BJ, _D), lambda j, i, *_: (j, 0)),
            pl.BlockSpec((_R, _BJ, _D), lambda j, i, *_: (i, j, 0)),
        ],
        out_specs=[
            pl.BlockSpec((_BJ, _D), lambda j, i, *_: (j, 0)),
            pl.BlockSpec((_R, _BJ, _D), lambda j, i, *_: (i, j, 0)),
        ],
    )
    output, new_buffer = pl.pallas_call(
        _body,
        grid_spec=grid_spec,
        out_shape=(
            jax.ShapeDtypeStruct((_B, _D), x.dtype),
            jax.ShapeDtypeStruct((_L, _B, _D), buffer.dtype),
        ),
    )(idx, cc, x, buffer)
    return output, new_buffer


# speedup vs baseline: 2.9562x; 1.0187x over previous
import jax
import jax.numpy as jnp
from jax.experimental import pallas as pl
from jax.experimental.pallas import tpu as pltpu

_L = 50
_B = 4096
_D = 128
_R = 7
_NJ = 1
_BJ = _B // _NJ


def _body(idx_ref, cc_ref, x_ref, buf_ref, out_ref, nbuf_ref):
    i = pl.program_id(1)
    r = idx_ref[0] - _R * i
    nbuf_ref[...] = buf_ref[...]
    hit = jnp.logical_and(r >= 0, r < _R)

    @pl.when(hit)
    def _write_slot():
        nbuf_ref[r] = x_ref[...]
        out_ref[...] = jnp.where(cc_ref[0] >= _L, buf_ref[r],
                                 jnp.zeros_like(x_ref))


def kernel(x, buffer, index, call_count):
    idx = jnp.asarray(index, jnp.int32).reshape(1)
    cc = jnp.asarray(call_count, jnp.int32).reshape(1)
    grid_spec = pltpu.PrefetchScalarGridSpec(
        num_scalar_prefetch=2,
        grid=(_NJ, pl.cdiv(_L, _R)),
        in_specs=[
            pl.BlockSpec((_BJ, _D), lambda j, i, *_: (j, 0)),
            pl.BlockSpec((_R, _BJ, _D), lambda j, i, *_: (i, j, 0)),
        ],
        out_specs=[
            pl.BlockSpec((_BJ, _D), lambda j, i, *_: (j, 0)),
            pl.BlockSpec((_R, _BJ, _D), lambda j, i, *_: (i, j, 0)),
        ],
    )
    output, new_buffer = pl.pallas_call(
        _body,
        grid_spec=grid_spec,
        out_shape=(
            jax.ShapeDtypeStruct((_B, _D), x.dtype),
            jax.ShapeDtypeStruct((_L, _B, _D), buffer.dtype),
        ),
        compiler_params=pltpu.CompilerParams(
            vmem_limit_bytes=110 * 1024 * 1024),
    )(idx, cc, x, buffer)
    return output, new_buffer


# speedup vs baseline: 2.9615x; 1.0018x over previous
import jax
import jax.numpy as jnp
from jax.experimental import pallas as pl
from jax.experimental.pallas import tpu as pltpu

_L = 50
_B = 4096
_D = 128
_R = 6
_NJ = 1
_BJ = _B // _NJ


def _body(idx_ref, cc_ref, x_ref, buf_ref, out_ref, nbuf_ref):
    i = pl.program_id(1)
    r = idx_ref[0] - _R * i
    nbuf_ref[...] = buf_ref[...]
    hit = jnp.logical_and(r >= 0, r < _R)

    @pl.when(hit)
    def _write_slot():
        nbuf_ref[r] = x_ref[...]
        out_ref[...] = jnp.where(cc_ref[0] >= _L, buf_ref[r],
                                 jnp.zeros_like(x_ref))


def kernel(x, buffer, index, call_count):
    idx = jnp.asarray(index, jnp.int32).reshape(1)
    cc = jnp.asarray(call_count, jnp.int32).reshape(1)
    grid_spec = pltpu.PrefetchScalarGridSpec(
        num_scalar_prefetch=2,
        grid=(_NJ, pl.cdiv(_L, _R)),
        in_specs=[
            pl.BlockSpec((_BJ, _D), lambda j, i, *_: (j, 0)),
            pl.BlockSpec((_R, _BJ, _D), lambda j, i, *_: (i, j, 0)),
        ],
        out_specs=[
            pl.BlockSpec((_BJ, _D), lambda j, i, *_: (j, 0)),
            pl.BlockSpec((_R, _BJ, _D), lambda j, i, *_: (i, j, 0)),
        ],
    )
    output, new_buffer = pl.pallas_call(
        _body,
        grid_spec=grid_spec,
        out_shape=(
            jax.ShapeDtypeStruct((_B, _D), x.dtype),
            jax.ShapeDtypeStruct((_L, _B, _D), buffer.dtype),
        ),
    )(idx, cc, x, buffer)
    return output, new_buffer
